# baseline (speedup 1.0000x reference)
import functools

import jax
import jax.numpy as jnp
from jax import lax
from jax.experimental import pallas as pl
from jax.experimental.pallas import tpu as pltpu
from jax.experimental.pallas import tpu_sc as plsc

N0, N1, N2 = 50000, 10000, 2048
E1, E2 = 160000, 32768
IN, HID, OUT, H = 256, 128, 128, 4

_f32 = jnp.float32
_i32 = jnp.int32


def _k1_body(x_ref, w_ref, att_ref, hs_ref, a_ref):
    hs = jnp.dot(x_ref[...].astype(jnp.bfloat16), w_ref[...].astype(jnp.bfloat16),
                 preferred_element_type=_f32)
    hs_ref[...] = hs
    a_ref[...] = jnp.dot(hs, att_ref[...], preferred_element_type=_f32, precision=lax.Precision.HIGHEST)


def _run_k1(xs, W1, attbd):
    B = 1000
    return pl.pallas_call(
        _k1_body,
        grid=(N1 // B,),
        in_specs=[
            pl.BlockSpec((B, IN), lambda i: (i, 0)),
            pl.BlockSpec((IN, H * HID), lambda i: (0, 0)),
            pl.BlockSpec((H * HID, 2 * H), lambda i: (0, 0)),
        ],
        out_specs=[
            pl.BlockSpec((B, H * HID), lambda i: (i, 0)),
            pl.BlockSpec((B, 2 * H), lambda i: (i, 0)),
        ],
        out_shape=[
            jax.ShapeDtypeStruct((N1, H * HID), _f32),
            jax.ShapeDtypeStruct((N1, 2 * H), _f32),
        ],
    )(xs, W1, attbd)


_K1C = 64
_NCH1 = E1 // _K1C
_ROWS1 = 632
_N1P = 16 * _ROWS1
_DR1 = _N1P // 128
_DR1P = 80

_LOG2E = 1.4426950408889634
_LN2_HI = 0.6931471805599453
_RND = 12582912.0


def _exp16(x):
    t = x * _LOG2E
    m = t + _RND
    kf = m - _RND
    r = (x - kf * _LN2_HI)
    p = r * (1.0 / 720.0) + (1.0 / 120.0)
    p = p * r + (1.0 / 24.0)
    p = p * r + (1.0 / 6.0)
    p = p * r + 0.5
    p = p * r + 1.0
    p = p * r + 1.0
    ki = kf.astype(_i32)
    two_k = lax.bitcast_convert_type(
        lax.shift_left(ki + 127, jnp.full((16,), 23, _i32)), _f32)
    return p * two_k


def _scale_and_onehot(rows_v, ohb_v, ev16, dmod16, nvec):
    iotas = [jnp.arange(j * 16, j * 16 + 16, dtype=_i32) for j in range(8)]
    for u in range(16):
        ev = jnp.full((16,), ev16[u], _f32)
        mv = jnp.full((16,), dmod16[u], _i32)
        base = nvec * 16 + u
        for j in range(8):
            rows_v[base, pl.ds(j * 16, 16)] = (
                rows_v[base, pl.ds(j * 16, 16)] * ev)
        for j in range(8):
            ohb_v[base, pl.ds(j * 16, 16)] = jnp.where(
                iotas[j] == mv, ev, jnp.zeros((16,), _f32))


def _sc1_body(hs_hbm, asrc_hbm, adst_hbm, src_hbm, dst_hbm, feat_hbm, den_hbm,
              asrc_v, adst_v, src_v, dst_v, gidx_v, didx_v, rows_v, ohb_v,
              feat_sh, den_sh):
    c = lax.axis_index("c")
    s = lax.axis_index("s")
    nch = jnp.where(s < 4, 157, 156)
    ebase = _K1C * (s * 156 + jnp.minimum(s, 4))
    zero16 = jnp.zeros((16,), _f32)

    for p in range(2):
        h = c * 2 + p
        pltpu.sync_copy(asrc_hbm.at[h], asrc_v)
        pltpu.sync_copy(adst_hbm.at[h], adst_v)
        for r in range(_K1C):
            for j in range(8):
                ohb_v[r, pl.ds(j * 16, 16)] = zero16
        r0 = s * _ROWS1
        for z in range(_ROWS1 // _K1C):
            pltpu.sync_copy(ohb_v, feat_sh.at[pl.ds(r0 + z * _K1C, _K1C)])
        rem = _ROWS1 % _K1C
        pltpu.sync_copy(ohb_v.at[pl.ds(0, rem)],
                        feat_sh.at[pl.ds(r0 + (_ROWS1 // _K1C) * _K1C, rem)])

        @pl.when(s < 10)
        def _():
            pltpu.sync_copy(ohb_v.at[pl.ds(0, 8)], den_sh.at[pl.ds(s * 8, 8)])

        plsc.subcore_barrier()

        def chunk(t, carry):
            base = pl.multiple_of(ebase + t * _K1C, 16)
            pltpu.sync_copy(src_hbm.at[pl.ds(base, _K1C)], src_v)
            pltpu.sync_copy(dst_hbm.at[pl.ds(base, _K1C)], dst_v)
            for i in range(_K1C // 16):
                sv = src_v[pl.ds(i * 16, 16)]
                dv = dst_v[pl.ds(i * 16, 16)]
                gidx_v[pl.ds(i * 16, 16)] = sv * H + h
                didx_v[pl.ds(i * 16, 16)] = lax.shift_right_logical(dv, 7)
            pltpu.sync_copy(hs_hbm.at[gidx_v], rows_v)
            for i in range(_K1C // 16):
                sv = src_v[pl.ds(i * 16, 16)]
                dv = dst_v[pl.ds(i * 16, 16)]
                asv = plsc.load_gather(asrc_v, [sv])
                adv = plsc.load_gather(adst_v, [dv])
                al = asv + adv
                al = jnp.where(al > 0, al, 0.2 * al)
                ev16 = _exp16(al)
                dmod16 = lax.bitwise_and(dv, jnp.full((16,), 127, _i32))
                _scale_and_onehot(rows_v, ohb_v, ev16, dmod16, i)
            pltpu.sync_copy(rows_v, feat_sh.at[dst_v], add=True)
            pltpu.sync_copy(ohb_v, den_sh.at[didx_v], add=True)
            return carry

        lax.fori_loop(0, nch, chunk, 0)
        plsc.subcore_barrier()
        pltpu.sync_copy(feat_sh.at[pl.ds(r0, _ROWS1)],
                        feat_hbm.at[h, pl.ds(r0, _ROWS1)])

        @pl.when(s < 10)
        def _():
            pltpu.sync_copy(den_sh.at[pl.ds(s * 8, 8)],
                            den_hbm.at[h, pl.ds(s * 8, 8)])

        plsc.subcore_barrier()


def _run_sc1(hs_flat, asrc_t, adst_t, src, dst):
    mesh = plsc.VectorSubcoreMesh(core_axis_name="c", subcore_axis_name="s")
    f = functools.partial(
        pl.kernel,
        mesh=mesh,
        out_type=[
            jax.ShapeDtypeStruct((H, _N1P, HID), _f32),
            jax.ShapeDtypeStruct((H, _DR1P, 128), _f32),
        ],
        scratch_types=[
            pltpu.VMEM((N1,), _f32),
            pltpu.VMEM((N1,), _f32),
            pltpu.VMEM((_K1C,), _i32),
            pltpu.VMEM((_K1C,), _i32),
            pltpu.VMEM((_K1C,), _i32),
            pltpu.VMEM((_K1C,), _i32),
            pltpu.VMEM((_K1C, HID), _f32),
            pltpu.VMEM((_K1C, 128), _f32),
            pltpu.VMEM_SHARED((_N1P, HID), _f32),
            pltpu.VMEM_SHARED((_DR1P, 128), _f32),
        ],
        compiler_params=pltpu.CompilerParams(needs_layout_passes=False),
    )(_sc1_body)
    return f(hs_flat, asrc_t, adst_t, src, dst)


def _k2a_body(f_ref, d_ref, b1_ref, s_ref):
    @pl.when(pl.program_id(0) == 0)
    def _():
        s_ref[...] = jnp.zeros_like(s_ref)

    blk = f_ref[...]
    den = d_ref[...]
    s1s, s2s = [], []
    for h in range(H):
        hv = blk[h] / (den[:, h:h + 1] + 1e-16) + b1_ref[...][h][None, :]
        s1s.append(jnp.sum(hv, axis=0, keepdims=True))
        s2s.append(jnp.sum(hv * hv, axis=0, keepdims=True))
    upd = jnp.stack([jnp.concatenate(s1s, axis=0),
                     jnp.concatenate(s2s, axis=0)])
    s_ref[...] += upd


def _run_k2a(feat, dent, bias1r):
    B = 1000
    return pl.pallas_call(
        _k2a_body,
        grid=(N1 // B,),
        in_specs=[
            pl.BlockSpec((H, B, HID), lambda i: (0, i, 0)),
            pl.BlockSpec((B, H), lambda i: (i, 0)),
            pl.BlockSpec((H, HID), lambda i: (0, 0)),
        ],
        out_specs=pl.BlockSpec((2, H, HID), lambda i: (0, 0, 0)),
        out_shape=jax.ShapeDtypeStruct((2, H, HID), _f32),
    )(feat, dent, bias1r)


def _k2b_body(f_ref, d_ref, s_ref, b1_ref, g_ref, be_ref, w2_ref, att2_ref,
              hs2_ref, a2_ref):
    mean = s_ref[0] / N1
    var = s_ref[1] / N1 - mean * mean
    scale = lax.rsqrt(var + 1e-5) * g_ref[...]
    den = d_ref[...]
    acc = jnp.zeros((N2, OUT), _f32)
    for h in range(H):
        hv = f_ref[h] / (den[:, h:h + 1] + 1e-16) + b1_ref[...][h][None, :]
        hn = (hv - mean[h][None, :]) * scale[h][None, :] + be_ref[...][h][None, :]
        he = jnp.where(hn > 0, hn, jnp.exp(jnp.minimum(hn, 0.0)) - 1.0)
        acc = acc + jnp.dot(he.astype(jnp.bfloat16),
                            w2_ref[h].astype(jnp.bfloat16),
                            preferred_element_type=_f32)
    hs2_ref[...] = acc
    a2_ref[...] = jnp.dot(acc, att2_ref[...], preferred_element_type=_f32, precision=lax.Precision.HIGHEST)


def _run_k2b(feat2, dent2, stats, bias1r, gammar, betar, W2r, att2):
    return pl.pallas_call(
        _k2b_body,
        in_specs=[pl.BlockSpec(memory_space=pltpu.VMEM)] * 8,
        out_specs=[pl.BlockSpec(memory_space=pltpu.VMEM)] * 2,
        out_shape=[
            jax.ShapeDtypeStruct((N2, OUT), _f32),
            jax.ShapeDtypeStruct((N2, 2), _f32),
        ],
    )(feat2, dent2, stats, bias1r, gammar, betar, W2r, att2)


_K2C = 64
_NCH2 = (E2 // 32) // _K2C
_ROWS2 = N2 // 16
_DR2 = N2 // 128


def _sc2_body(hs_hbm, asrc_hbm, adst_hbm, src_hbm, dst_hbm, feat_hbm, den_hbm,
              asrc_v, adst_v, src_v, dst_v, didx_v, rows_v, ohb_v,
              feat_sh, den_sh):
    c = lax.axis_index("c")
    s = lax.axis_index("s")
    w = s * 2 + c
    ebase = w * (E2 // 32)
    pltpu.sync_copy(asrc_hbm, asrc_v)
    pltpu.sync_copy(adst_hbm, adst_v)
    zero16 = jnp.zeros((16,), _f32)
    for r in range(_K2C):
        for j in range(8):
            ohb_v[r, pl.ds(j * 16, 16)] = zero16
    r0 = s * _ROWS2
    for z in range(_ROWS2 // _K2C):
        pltpu.sync_copy(ohb_v, feat_sh.at[pl.ds(r0 + z * _K2C, _K2C)])

    @pl.when(s < 2)
    def _():
        pltpu.sync_copy(ohb_v.at[pl.ds(0, 8)], den_sh.at[pl.ds(s * 8, 8)])

    plsc.subcore_barrier()

    def chunk(t, carry):
        base = pl.multiple_of(ebase + t * _K2C, 16)
        pltpu.sync_copy(src_hbm.at[pl.ds(base, _K2C)], src_v)
        pltpu.sync_copy(dst_hbm.at[pl.ds(base, _K2C)], dst_v)
        for i in range(_K2C // 16):
            dv = dst_v[pl.ds(i * 16, 16)]
            didx_v[pl.ds(i * 16, 16)] = lax.shift_right_logical(dv, 7)
        pltpu.sync_copy(hs_hbm.at[src_v], rows_v)
        for i in range(_K2C // 16):
            sv = src_v[pl.ds(i * 16, 16)]
            dv = dst_v[pl.ds(i * 16, 16)]
            asv = plsc.load_gather(asrc_v, [sv])
            adv = plsc.load_gather(adst_v, [dv])
            al = asv + adv
            al = jnp.where(al > 0, al, 0.2 * al)
            ev16 = _exp16(al)
            dmod16 = lax.bitwise_and(dv, jnp.full((16,), 127, _i32))
            _scale_and_onehot(rows_v, ohb_v, ev16, dmod16, i)
        pltpu.sync_copy(rows_v, feat_sh.at[dst_v], add=True)
        pltpu.sync_copy(ohb_v.at[pl.ds(0, _K2C)], den_sh.at[didx_v], add=True)
        return carry

    lax.fori_loop(0, _NCH2, chunk, 0)
    plsc.subcore_barrier()
    pltpu.sync_copy(feat_sh.at[pl.ds(r0, _ROWS2)],
                    feat_hbm.at[c, pl.ds(r0, _ROWS2)])

    @pl.when(s < 2)
    def _():
        pltpu.sync_copy(den_sh.at[pl.ds(s * 8, 8)],
                        den_hbm.at[c, pl.ds(s * 8, 8)])


def _run_sc2(hs2, asrc2, adst2, src, dst):
    mesh = plsc.VectorSubcoreMesh(core_axis_name="c", subcore_axis_name="s")
    f = functools.partial(
        pl.kernel,
        mesh=mesh,
        out_type=[
            jax.ShapeDtypeStruct((2, N2, OUT), _f32),
            jax.ShapeDtypeStruct((2, _DR2, 128), _f32),
        ],
        scratch_types=[
            pltpu.VMEM((N2,), _f32),
            pltpu.VMEM((N2,), _f32),
            pltpu.VMEM((_K2C,), _i32),
            pltpu.VMEM((_K2C,), _i32),
            pltpu.VMEM((_K2C,), _i32),
            pltpu.VMEM((_K2C, OUT), _f32),
            pltpu.VMEM((_K2C, 128), _f32),
            pltpu.VMEM_SHARED((N2, OUT), _f32),
            pltpu.VMEM_SHARED((_DR2, 128), _f32),
        ],
        compiler_params=pltpu.CompilerParams(needs_layout_passes=False),
    )(_sc2_body)
    return f(hs2, asrc2, adst2, src, dst)


def _k4_body(p_ref, d_ref, b2_ref, o_ref):
    num = p_ref[0] + p_ref[1]
    d = d_ref[...]
    den = d[:, 0:1] + d[:, 1:2]
    o_ref[...] = num / (den + 1e-16) + b2_ref[...]


def _run_k4(parts, dent2, bias2):
    return pl.pallas_call(
        _k4_body,
        in_specs=[pl.BlockSpec(memory_space=pltpu.VMEM)] * 3,
        out_specs=pl.BlockSpec(memory_space=pltpu.VMEM),
        out_shape=jax.ShapeDtypeStruct((N2, OUT), _f32),
    )(parts, dent2, bias2)


def kernel(x, edge_index1, edge_index2, W1, att_src1, att_dst1, bias1,
           gamma, beta, W2, att_src2, att_dst2, bias2):
    xs = x[:N1]
    src1 = edge_index1[0].astype(_i32)
    dst1 = edge_index1[1].astype(_i32)
    src2 = edge_index2[0].astype(_i32)
    dst2 = edge_index2[1].astype(_i32)

    eye = jnp.eye(H, dtype=_f32)
    asrc_bd = (eye[:, None, :] * att_src1[:, :, None]).reshape(H * HID, H)
    adst_bd = (eye[:, None, :] * att_dst1[:, :, None]).reshape(H * HID, H)
    attbd = jnp.concatenate([asrc_bd, adst_bd], axis=1)

    hs, acat = _run_k1(xs, W1, attbd)
    hs_flat = hs.reshape(N1 * H, HID)
    asrc_t = acat[:, :H].T
    adst_t = acat[:, H:].T

    feat, den = _run_sc1(hs_flat, asrc_t, adst_t, src1, dst1)
    feat = feat[:, :N1]
    dent = den.reshape(H, _DR1P * 128)[:, :N1].T

    bias1r = bias1.reshape(H, HID)
    stats = _run_k2a(feat, dent, bias1r)
    hs2, acat2 = _run_k2b(feat[:, :N2], dent[:N2], stats, bias1r,
                          gamma.reshape(H, HID), beta.reshape(H, HID),
                          W2.reshape(H, HID, OUT),
                          jnp.stack([att_src2[0], att_dst2[0]], axis=1))

    parts, den2 = _run_sc2(hs2, acat2[:, 0], acat2[:, 1], src2, dst2)
    dent2 = den2.reshape(2, N2).T
    return _run_k4(parts, dent2, bias2)


# speedup vs baseline: 1.9674x; 1.9674x over previous
import functools

import jax
import jax.numpy as jnp
from jax import lax
from jax.experimental import pallas as pl
from jax.experimental.pallas import tpu as pltpu
from jax.experimental.pallas import tpu_sc as plsc

N0, N1, N2 = 50000, 10000, 2048
E1, E2 = 160000, 32768
IN, HID, OUT, H = 256, 128, 128, 4

_f32 = jnp.float32
_i32 = jnp.int32

_LOG2E = 1.4426950408889634
_LN2 = 0.6931471805599453
_RND = 12582912.0


def _exp16(x):
    t = x * _LOG2E
    m = t + _RND
    kf = m - _RND
    r = x - kf * _LN2
    p = r * (1.0 / 720.0) + (1.0 / 120.0)
    p = p * r + (1.0 / 24.0)
    p = p * r + (1.0 / 6.0)
    p = p * r + 0.5
    p = p * r + 1.0
    p = p * r + 1.0
    ki = kf.astype(_i32)
    two_k = lax.bitcast_convert_type(
        lax.shift_left(ki + 127, jnp.full((16,), 23, _i32)), _f32)
    return p * two_k


def _k1_body(x_ref, w_ref, att_ref, hs_ref, a_ref):
    hs = jnp.dot(x_ref[...].astype(jnp.bfloat16),
                 w_ref[...].astype(jnp.bfloat16),
                 preferred_element_type=_f32)
    hs_ref[...] = hs
    a_ref[...] = jnp.dot(hs, att_ref[...], preferred_element_type=_f32,
                         precision=lax.Precision.HIGHEST)


def _run_k1(xs, W1, attbd):
    B = 1000
    return pl.pallas_call(
        _k1_body,
        grid=(N1 // B,),
        in_specs=[
            pl.BlockSpec((B, IN), lambda i: (i, 0)),
            pl.BlockSpec((IN, H * HID), lambda i: (0, 0)),
            pl.BlockSpec((H * HID, 2 * H), lambda i: (0, 0)),
        ],
        out_specs=[
            pl.BlockSpec((B, H * HID), lambda i: (i, 0)),
            pl.BlockSpec((B, 2 * H), lambda i: (i, 0)),
        ],
        out_shape=[
            jax.ShapeDtypeStruct((N1, H * HID), _f32),
            jax.ShapeDtypeStruct((N1, 2 * H), _f32),
        ],
    )(xs, W1, attbd)


_KC = 64
_ROWS1 = 632
_N1P = 16 * _ROWS1
_ROWS2 = N2 // 16


def _edge_pass(h_mul, h_off, nch, npairs, has_tail, ebase,
               hs_hbm, src_hbm, dst_hbm, feat_sh,
               asrc_v, adst_v, den_v,
               src0, dst0, gidx0, sdst0, rows0,
               src1, dst1, gidx1, sdst1, rows1,
               isem0, isem1, gsem0, gsem1, ssem0, ssem1):
    def issue_idx(t, sb, db, isem):
        base = pl.multiple_of(ebase + t * _KC, 16)
        pltpu.async_copy(src_hbm.at[pl.ds(base, _KC)], sb, isem)
        pltpu.async_copy(dst_hbm.at[pl.ds(base, _KC)], db, isem)

    def wait_idx(sb, db, isem):
        pltpu.make_async_copy(src_hbm.at[pl.ds(0, _KC)], sb, isem).wait()
        pltpu.make_async_copy(dst_hbm.at[pl.ds(0, _KC)], db, isem).wait()

    def compute_gidx(sb, gb):
        for i in range(_KC // 16):
            gb[pl.ds(i * 16, 16)] = sb[pl.ds(i * 16, 16)] * h_mul + h_off

    def issue_gather(gb, rb, gsem):
        pltpu.async_copy(hs_hbm.at[gb], rb, gsem)

    def wait_gather(gb, rb, gsem):
        pltpu.make_async_copy(hs_hbm.at[gb], rb, gsem).wait()

    def compute(sb, db, sdb, rb):
        for i in range(_KC // 16):
            sv = sb[pl.ds(i * 16, 16)]
            dv = db[pl.ds(i * 16, 16)]
            sdb[pl.ds(i * 16, 16)] = dv
            asv = plsc.load_gather(asrc_v, [sv])
            adv = plsc.load_gather(adst_v, [dv])
            al = asv + adv
            al = jnp.where(al > 0, al, 0.2 * al)
            ev16 = _exp16(al)
            plsc.addupdate_scatter(den_v, [dv], ev16)
            for u in range(16):
                ev = jnp.full((16,), ev16[u], _f32)
                r = i * 16 + u
                for j in range(8):
                    rb[r, pl.ds(j * 16, 16)] = rb[r, pl.ds(j * 16, 16)] * ev

    def issue_scatter(rb, sdb, ssem):
        pltpu.async_copy(rb, feat_sh.at[sdb], ssem, add=True)

    def wait_scatter(rb, sdb, ssem):
        pltpu.make_async_copy(rb, feat_sh.at[sdb], ssem).wait()

    issue_idx(0, src0, dst0, isem0)
    issue_idx(1, src1, dst1, isem1)
    wait_idx(src0, dst0, isem0)
    compute_gidx(src0, gidx0)
    issue_gather(gidx0, rows0, gsem0)

    def pair(t2, carry):
        a = 2 * t2
        b = a + 1
        wait_gather(gidx0, rows0, gsem0)

        @pl.when(t2 > 0)
        def _():
            wait_scatter(rows1, sdst1, ssem1)

        wait_idx(src1, dst1, isem1)
        compute_gidx(src1, gidx1)
        issue_gather(gidx1, rows1, gsem1)

        compute(src0, dst0, sdst0, rows0)
        issue_scatter(rows0, sdst0, ssem0)

        @pl.when(a + 2 < nch)
        def _():
            issue_idx(a + 2, src0, dst0, isem0)

        wait_gather(gidx1, rows1, gsem1)
        compute(src1, dst1, sdst1, rows1)
        issue_scatter(rows1, sdst1, ssem1)

        @pl.when(b + 2 < nch)
        def _():
            issue_idx(b + 2, src1, dst1, isem1)

        @pl.when(a + 2 < nch)
        def _():
            wait_idx(src0, dst0, isem0)
            compute_gidx(src0, gidx0)

        wait_scatter(rows0, sdst0, ssem0)

        @pl.when(a + 2 < nch)
        def _():
            issue_gather(gidx0, rows0, gsem0)

        return carry

    lax.fori_loop(0, npairs, pair, 0)

    if has_tail:
        @pl.when(nch > 2 * npairs)
        def _():
            wait_gather(gidx0, rows0, gsem0)
            compute(src0, dst0, sdst0, rows0)
            issue_scatter(rows0, sdst0, ssem0)
            wait_scatter(rows0, sdst0, ssem0)

    wait_scatter(rows1, sdst1, ssem1)


def _zero_vmem_rows(rb, nrows):
    z = jnp.zeros((16,), _f32)
    for r in range(nrows):
        for j in range(8):
            rb[r, pl.ds(j * 16, 16)] = z


def _zero_den(den_v, n):
    def body(t, carry):
        den_v[pl.ds(pl.multiple_of(t * 16, 16), 16)] = jnp.zeros((16,), _f32)
        return carry
    lax.fori_loop(0, n // 16, body, 0)


def _sc1_body(hs_hbm, asrc_hbm, adst_hbm, src_hbm, dst_hbm, feat_hbm, den_hbm,
              asrc_v, adst_v, den_v,
              src0, dst0, gidx0, sdst0, rows0,
              src1, dst1, gidx1, sdst1, rows1,
              isem0, isem1, gsem0, gsem1, ssem0, ssem1, feat_sh):
    c = lax.axis_index("c")
    s = lax.axis_index("s")
    nch = jnp.where(s < 4, 157, 156)
    ebase = _KC * (s * 156 + jnp.minimum(s, 4))
    r0 = s * _ROWS1

    for p in range(2):
        h = c * 2 + p
        pltpu.sync_copy(asrc_hbm.at[h], asrc_v)
        pltpu.sync_copy(adst_hbm.at[h], adst_v)
        _zero_den(den_v, _N1P)
        _zero_vmem_rows(rows0, _KC)
        for z in range(_ROWS1 // _KC):
            pltpu.sync_copy(rows0, feat_sh.at[pl.ds(r0 + z * _KC, _KC)])
        rem = _ROWS1 % _KC
        pltpu.sync_copy(rows0.at[pl.ds(0, rem)],
                        feat_sh.at[pl.ds(r0 + (_ROWS1 // _KC) * _KC, rem)])
        plsc.subcore_barrier()

        _edge_pass(H, h, nch, 78, True, ebase,
                   hs_hbm, src_hbm, dst_hbm, feat_sh,
                   asrc_v, adst_v, den_v,
                   src0, dst0, gidx0, sdst0, rows0,
                   src1, dst1, gidx1, sdst1, rows1,
                   isem0, isem1, gsem0, gsem1, ssem0, ssem1)

        plsc.subcore_barrier()
        pltpu.sync_copy(feat_sh.at[pl.ds(r0, _ROWS1)],
                        feat_hbm.at[h, pl.ds(r0, _ROWS1)])
        pltpu.sync_copy(den_v, den_hbm.at[h, s])
        plsc.subcore_barrier()


def _run_sc1(hs_flat, asrc_t, adst_t, src, dst):
    mesh = plsc.VectorSubcoreMesh(core_axis_name="c", subcore_axis_name="s")
    f = functools.partial(
        pl.kernel,
        mesh=mesh,
        out_type=[
            jax.ShapeDtypeStruct((H, _N1P, HID), _f32),
            jax.ShapeDtypeStruct((H, 16, _N1P), _f32),
        ],
        scratch_types=[
            pltpu.VMEM((N1,), _f32),
            pltpu.VMEM((N1,), _f32),
            pltpu.VMEM((_N1P,), _f32),
            pltpu.VMEM((_KC,), _i32),
            pltpu.VMEM((_KC,), _i32),
            pltpu.VMEM((_KC,), _i32),
            pltpu.VMEM((_KC,), _i32),
            pltpu.VMEM((_KC, HID), _f32),
            pltpu.VMEM((_KC,), _i32),
            pltpu.VMEM((_KC,), _i32),
            pltpu.VMEM((_KC,), _i32),
            pltpu.VMEM((_KC,), _i32),
            pltpu.VMEM((_KC, HID), _f32),
            pltpu.SemaphoreType.DMA,
            pltpu.SemaphoreType.DMA,
            pltpu.SemaphoreType.DMA,
            pltpu.SemaphoreType.DMA,
            pltpu.SemaphoreType.DMA,
            pltpu.SemaphoreType.DMA,
            pltpu.VMEM_SHARED((_N1P, HID), _f32),
        ],
        compiler_params=pltpu.CompilerParams(needs_layout_passes=False),
    )(_sc1_body)
    return f(hs_flat, asrc_t, adst_t, src, dst)


def _sc2_body(hs_hbm, asrc_hbm, adst_hbm, src_hbm, dst_hbm, feat_hbm, den_hbm,
              asrc_v, adst_v, den_v,
              src0, dst0, gidx0, sdst0, rows0,
              src1, dst1, gidx1, sdst1, rows1,
              isem0, isem1, gsem0, gsem1, ssem0, ssem1, feat_sh):
    c = lax.axis_index("c")
    s = lax.axis_index("s")
    w = s * 2 + c
    nch = E2 // 32 // _KC
    ebase = w * (E2 // 32)
    pltpu.sync_copy(asrc_hbm, asrc_v)
    pltpu.sync_copy(adst_hbm, adst_v)
    _zero_den(den_v, N2)
    _zero_vmem_rows(rows0, _KC)
    r0 = s * _ROWS2
    for z in range(_ROWS2 // _KC):
        pltpu.sync_copy(rows0, feat_sh.at[pl.ds(r0 + z * _KC, _KC)])
    plsc.subcore_barrier()

    _edge_pass(1, 0, nch, nch // 2, False, ebase,
               hs_hbm, src_hbm, dst_hbm, feat_sh,
               asrc_v, adst_v, den_v,
               src0, dst0, gidx0, sdst0, rows0,
               src1, dst1, gidx1, sdst1, rows1,
               isem0, isem1, gsem0, gsem1, ssem0, ssem1)

    plsc.subcore_barrier()
    pltpu.sync_copy(feat_sh.at[pl.ds(r0, _ROWS2)],
                    feat_hbm.at[c, pl.ds(r0, _ROWS2)])
    pltpu.sync_copy(den_v, den_hbm.at[c, s])


def _run_sc2(hs2, asrc2, adst2, src, dst):
    mesh = plsc.VectorSubcoreMesh(core_axis_name="c", subcore_axis_name="s")
    f = functools.partial(
        pl.kernel,
        mesh=mesh,
        out_type=[
            jax.ShapeDtypeStruct((2, N2, OUT), _f32),
            jax.ShapeDtypeStruct((2, 16, N2), _f32),
        ],
        scratch_types=[
            pltpu.VMEM((N2,), _f32),
            pltpu.VMEM((N2,), _f32),
            pltpu.VMEM((N2,), _f32),
            pltpu.VMEM((_KC,), _i32),
            pltpu.VMEM((_KC,), _i32),
            pltpu.VMEM((_KC,), _i32),
            pltpu.VMEM((_KC,), _i32),
            pltpu.VMEM((_KC, OUT), _f32),
            pltpu.VMEM((_KC,), _i32),
            pltpu.VMEM((_KC,), _i32),
            pltpu.VMEM((_KC,), _i32),
            pltpu.VMEM((_KC,), _i32),
            pltpu.VMEM((_KC, OUT), _f32),
            pltpu.SemaphoreType.DMA,
            pltpu.SemaphoreType.DMA,
            pltpu.SemaphoreType.DMA,
            pltpu.SemaphoreType.DMA,
            pltpu.SemaphoreType.DMA,
            pltpu.SemaphoreType.DMA,
            pltpu.VMEM_SHARED((N2, OUT), _f32),
        ],
        compiler_params=pltpu.CompilerParams(needs_layout_passes=False),
    )(_sc2_body)
    return f(hs2, asrc2, adst2, src, dst)


_DN = (((0,), (0,)), ((), ()))


def _dcol(dh):
    ones = jnp.ones((16, 1), _f32)
    return lax.dot_general(dh, ones, dimension_numbers=_DN,
                           preferred_element_type=_f32,
                           precision=lax.Precision.HIGHEST)


def _k2a_body(f_ref, d_ref, b1_ref, s_ref):
    @pl.when(pl.program_id(0) == 0)
    def _():
        s_ref[...] = jnp.zeros_like(s_ref)

    blk = f_ref[...]
    i = pl.program_id(0)
    B = blk.shape[1]
    rid = lax.broadcasted_iota(_i32, (B, HID), 0) + i * B
    s1s, s2s = [], []
    for h in range(H):
        den = _dcol(d_ref[...][h])
        hv = blk[h] / (den + 1e-16) + b1_ref[...][h][None, :]
        hv = jnp.where(rid < N1, hv, 0.0)
        s1s.append(jnp.sum(hv, axis=0, keepdims=True))
        s2s.append(jnp.sum(hv * hv, axis=0, keepdims=True))
    upd = jnp.stack([jnp.concatenate(s1s, axis=0),
                     jnp.concatenate(s2s, axis=0)])
    s_ref[...] += upd


def _run_k2a(feat, denp, bias1r):
    B = 128
    return pl.pallas_call(
        _k2a_body,
        grid=(_N1P // B,),
        in_specs=[
            pl.BlockSpec((H, B, HID), lambda i: (0, i, 0)),
            pl.BlockSpec((H, 16, B), lambda i: (0, 0, i)),
            pl.BlockSpec((H, HID), lambda i: (0, 0)),
        ],
        out_specs=pl.BlockSpec((2, H, HID), lambda i: (0, 0, 0)),
        out_shape=jax.ShapeDtypeStruct((2, H, HID), _f32),
    )(feat, denp, bias1r)


def _k2b_body(f_ref, d_ref, s_ref, b1_ref, g_ref, be_ref, w2_ref, att2_ref,
              hs2_ref, a2_ref):
    mean = s_ref[0] / N1
    var = s_ref[1] / N1 - mean * mean
    scale = lax.rsqrt(var + 1e-5) * g_ref[...]
    acc = jnp.zeros((N2, OUT), _f32)
    for h in range(H):
        den = _dcol(d_ref[...][h])
        hv = f_ref[h] / (den + 1e-16) + b1_ref[...][h][None, :]
        hn = (hv - mean[h][None, :]) * scale[h][None, :] + be_ref[...][h][None, :]
        he = jnp.where(hn > 0, hn, jnp.exp(jnp.minimum(hn, 0.0)) - 1.0)
        acc = acc + jnp.dot(he.astype(jnp.bfloat16),
                            w2_ref[h].astype(jnp.bfloat16),
                            preferred_element_type=_f32)
    hs2_ref[...] = acc
    a2_ref[...] = jnp.dot(acc, att2_ref[...], preferred_element_type=_f32,
                          precision=lax.Precision.HIGHEST)


def _run_k2b(feat2, denp2, stats, bias1r, gammar, betar, W2r, att2):
    return pl.pallas_call(
        _k2b_body,
        in_specs=[pl.BlockSpec(memory_space=pltpu.VMEM)] * 8,
        out_specs=[pl.BlockSpec(memory_space=pltpu.VMEM)] * 2,
        out_shape=[
            jax.ShapeDtypeStruct((N2, OUT), _f32),
            jax.ShapeDtypeStruct((N2, 2), _f32),
        ],
    )(feat2, denp2, stats, bias1r, gammar, betar, W2r, att2)


def _k4_body(p_ref, d_ref, b2_ref, o_ref):
    num = p_ref[0] + p_ref[1]
    dh = d_ref[0] + d_ref[1]
    den = _dcol(dh)
    o_ref[...] = num / (den + 1e-16) + b2_ref[...]


def _run_k4(parts, denp2, bias2):
    return pl.pallas_call(
        _k4_body,
        in_specs=[pl.BlockSpec(memory_space=pltpu.VMEM)] * 3,
        out_specs=pl.BlockSpec(memory_space=pltpu.VMEM),
        out_shape=jax.ShapeDtypeStruct((N2, OUT), _f32),
    )(parts, denp2, bias2)


def kernel(x, edge_index1, edge_index2, W1, att_src1, att_dst1, bias1,
           gamma, beta, W2, att_src2, att_dst2, bias2):
    xs = x[:N1]
    src1 = edge_index1[0].astype(_i32)
    dst1 = edge_index1[1].astype(_i32)
    src2 = edge_index2[0].astype(_i32)
    dst2 = edge_index2[1].astype(_i32)

    eye = jnp.eye(H, dtype=_f32)
    asrc_bd = (eye[:, None, :] * att_src1[:, :, None]).reshape(H * HID, H)
    adst_bd = (eye[:, None, :] * att_dst1[:, :, None]).reshape(H * HID, H)
    attbd = jnp.concatenate([asrc_bd, adst_bd], axis=1)

    hs, acat = _run_k1(xs, W1, attbd)
    hs_flat = hs.reshape(N1 * H, HID)
    asrc_t = acat[:, :H].T
    adst_t = acat[:, H:].T

    feat, denp = _run_sc1(hs_flat, asrc_t, adst_t, src1, dst1)

    bias1r = bias1.reshape(H, HID)
    stats = _run_k2a(feat, denp, bias1r)
    hs2, acat2 = _run_k2b(feat[:, :N2], denp[:, :, :N2], stats, bias1r,
                          gamma.reshape(H, HID), beta.reshape(H, HID),
                          W2.reshape(H, HID, OUT),
                          jnp.stack([att_src2[0], att_dst2[0]], axis=1))

    parts, denp2 = _run_sc2(hs2, acat2[:, 0], acat2[:, 1], src2, dst2)
    return _run_k4(parts, denp2, bias2)
